# baseline (speedup 1.0000x reference)
import functools

import jax
import jax.numpy as jnp
from jax import lax
from jax.experimental import pallas as pl
from jax.experimental.pallas import tpu as pltpu
from jax.experimental.pallas import tpu_sc as plsc

_INTERPRET_TC = False


def _proj_body(x_ref, wl_ref, bl_ref, wr_ref, br_ref, xl_ref, xr_ref):
    x = x_ref[...]
    xl_ref[...] = (jnp.dot(x, wl_ref[...], preferred_element_type=jnp.float32)
                   + bl_ref[...][None, :])
    xr_ref[...] = (jnp.dot(x, wr_ref[...], preferred_element_type=jnp.float32)
                   + br_ref[...][None, :])


def _eamask_body(ea_ref, s_ref, d_ref, t16_ref, eam_ref):
    ea = ea_ref[...]
    s = s_ref[0]
    d = d_ref[0]
    keep = (s != d).astype(jnp.float32)
    b = ea.shape[0]
    col = lax.broadcasted_iota(jnp.int32, (b, 128), 1)
    tiled = jnp.dot(ea, t16_ref[...], preferred_element_type=jnp.float32)
    row = jnp.where(col % 32 == 16, 1.0, tiled)
    kmask = (col // 32 == d % 4).astype(jnp.float32)
    eam_ref[...] = row * keep * kmask


def _eemb_body(ea_ref, we_ref, eemb_ref):
    eemb_ref[...] = jnp.dot(ea_ref[...], we_ref[...],
                            preferred_element_type=jnp.float32)


def _edge_body(xl_ref, xr_ref, ee_ref, s_ref, d_ref, aatt_ref, rrep_ref,
               t8_ref, w_ref, exrow_ref):
    xl = xl_ref[...]
    m = xl + xr_ref[...] + ee_ref[...]
    mlr = jnp.where(m > 0, m, 0.2 * m)
    alpha = jnp.dot(mlr, aatt_ref[...], preferred_element_type=jnp.float32)
    s = s_ref[0]
    d = d_ref[0]
    keep = (s != d).astype(jnp.float32)
    ex = jnp.exp(alpha) * keep
    w_ref[...] = jnp.dot(ex, rrep_ref[...],
                         preferred_element_type=jnp.float32) * xl
    b = xl.shape[0]
    col = lax.broadcasted_iota(jnp.int32, (b, 128), 1)
    tiledex = jnp.dot(ex, t8_ref[...], preferred_element_type=jnp.float32)
    exrow_ref[...] = tiledex * (col // 8 == d % 16).astype(jnp.float32)


def _final_body(lacc_ref, wacc_ref, den_ref, xl_ref, xr_ref, we_ref,
                aatt_ref, rrep_ref, bias_ref, out_ref):
    lsum = lacc_ref[0] + lacc_ref[1]
    deg = lsum[:, 16:17]
    la = lsum[:, 0:16] / jnp.maximum(deg, 1.0)
    el = jnp.dot(la, we_ref[...], preferred_element_type=jnp.float32)
    xl = xl_ref[...]
    m = xl + xr_ref[...] + el
    mlr = jnp.where(m > 0, m, 0.2 * m)
    al = jnp.dot(mlr, aatt_ref[...], preferred_element_type=jnp.float32)
    exl = jnp.exp(al)
    densum = den_ref[0] + den_ref[1] + exl
    rrep = rrep_ref[...]
    num = (wacc_ref[0] + wacc_ref[1]
           + jnp.dot(exl, rrep, preferred_element_type=jnp.float32) * xl)
    denrep = jnp.dot(densum, rrep, preferred_element_type=jnp.float32) + 1e-30
    out_ref[...] = num / denrep + bias_ref[...][None, :]


_NC, _NS = 2, 16
_NW = _NC * _NS
_CH = 80


def _sc_gather_body(n_iters, src_hbm, dst_hbm, xl_hbm, xr_hbm, eam_hbm, z_hbm,
                    xls_hbm, xrd_hbm, lacc_hbm,
                    sidx, didx, didx4, rows_a, rows_b, ea_rows,
                    sem_i, sem_g, sem_w, acc_sp):
    cid = lax.axis_index("c")
    sid = lax.axis_index("s")
    wid = sid * _NC + cid
    epw = n_iters * _CH

    @pl.when(sid == 0)
    def _init():
        pltpu.sync_copy(z_hbm, acc_sp)
    plsc.subcore_barrier()

    def issue_idx(ci, b):
        base = wid * epw + ci * _CH
        return (
            pltpu.async_copy(src_hbm.at[pl.ds(base, _CH)], sidx[b],
                             sem_i[b][0]),
            pltpu.async_copy(dst_hbm.at[pl.ds(base, _CH)], didx[b],
                             sem_i[b][1]),
        )

    def issue_gathers(ci, b):
        base = wid * epw + ci * _CH
        return (
            pltpu.async_copy(xl_hbm.at[sidx[b]], rows_a[b], sem_g[b][0]),
            pltpu.async_copy(xr_hbm.at[didx[b]], rows_b[b], sem_g[b][1]),
            pltpu.async_copy(eam_hbm.at[pl.ds(base, _CH)], ea_rows[b],
                             sem_g[b][2]),
        )

    def issue_drain(ci, b):
        base = wid * epw + ci * _CH
        w1 = pltpu.async_copy(rows_a[b], xls_hbm.at[pl.ds(base, _CH)],
                              sem_w[b][0])
        w2 = pltpu.async_copy(rows_b[b], xrd_hbm.at[pl.ds(base, _CH)],
                              sem_w[b][1])
        for g in range(_CH // 16):
            didx4[b][pl.ds(g * 16, 16)] = didx[b][pl.ds(g * 16, 16)] >> 2
        w3 = pltpu.async_copy(ea_rows[b], acc_sp.at[didx4[b]], sem_w[b][2],
                              add=True)
        return (w1, w2, w3)

    def wait_all(handles):
        for h in handles:
            h.wait()

    def do_pair(c0, c1):
        hi0 = issue_idx(c0, 0)
        hi1 = issue_idx(c1, 1)
        wait_all(hi0)
        hg0 = issue_gathers(c0, 0)
        wait_all(hi1)
        hg1 = issue_gathers(c1, 1)
        wait_all(hg0)
        hw0 = issue_drain(c0, 0)
        wait_all(hg1)
        hw1 = issue_drain(c1, 1)
        wait_all(hw0)
        wait_all(hw1)

    def pairbody(i, _):
        do_pair(2 * i, 2 * i + 1)
        return ()

    lax.fori_loop(0, n_iters // 2, pairbody, (), unroll=False)
    if n_iters % 2 == 1:
        c = n_iters - 1
        wait_all(issue_idx(c, 0))
        wait_all(issue_gathers(c, 0))
        wait_all(issue_drain(c, 0))

    plsc.subcore_barrier()

    @pl.when(sid == 0)
    def _writeout():
        pltpu.sync_copy(acc_sp, lacc_hbm.at[cid])


def _sc_scatter_body(n_iters, dst_hbm, w_hbm, exrow_hbm, z_hbm, zd_hbm,
                     wacc_hbm, dacc_hbm,
                     didx, didxc, didx16, w_rows, ex_rows, sem_i, sem_s,
                     wacc_sp, den_sp):
    cid = lax.axis_index("c")
    sid = lax.axis_index("s")
    wid = sid * _NC + cid
    epw = n_iters * _CH

    @pl.when(sid == 0)
    def _init_w():
        pltpu.sync_copy(z_hbm, wacc_sp)

    @pl.when(sid == 1)
    def _init_d():
        pltpu.sync_copy(zd_hbm, den_sp)
    plsc.subcore_barrier()

    def issue_loads(ci, b):
        base = wid * epw + ci * _CH
        return (
            pltpu.async_copy(dst_hbm.at[pl.ds(base, _CH)], didx[b],
                             sem_i[b][0]),
            pltpu.async_copy(w_hbm.at[pl.ds(base, _CH)], w_rows[b],
                             sem_i[b][1]),
            pltpu.async_copy(exrow_hbm.at[pl.ds(base, _CH)], ex_rows[b],
                             sem_i[b][2]),
        )

    def issue_scatters(b):
        for g in range(_CH // 16):
            didx16[b][pl.ds(g * 16, 16)] = didx[b][pl.ds(g * 16, 16)] >> 4
        return (
            pltpu.async_copy(w_rows[b], wacc_sp.at[didx[b]], sem_s[b][0],
                             add=True),
            pltpu.async_copy(ex_rows[b], den_sp.at[didx16[b]], sem_s[b][1],
                             add=True),
        )

    def wait_all(handles):
        for h in handles:
            h.wait()

    def do_pair(c0, c1):
        hl0 = issue_loads(c0, 0)
        hl1 = issue_loads(c1, 1)
        wait_all(hl0)
        hs0 = issue_scatters(0)
        wait_all(hl1)
        hs1 = issue_scatters(1)
        wait_all(hs0)
        wait_all(hs1)

    def pairbody(i, _):
        do_pair(2 * i, 2 * i + 1)
        return ()

    lax.fori_loop(0, n_iters // 2, pairbody, (), unroll=False)
    if n_iters % 2 == 1:
        wait_all(issue_loads(n_iters - 1, 0))
        wait_all(issue_scatters(0))
    plsc.subcore_barrier()

    rows_per_tile = wacc_sp.shape[0] // 10

    @pl.when(sid < 10)
    def _writeout_w():
        r0 = sid * rows_per_tile
        pltpu.sync_copy(wacc_sp.at[pl.ds(r0, rows_per_tile)],
                        wacc_hbm.at[cid, pl.ds(r0, rows_per_tile)])

    @pl.when(sid == 10)
    def _writeout_d():
        pltpu.sync_copy(den_sp, dacc_hbm.at[cid])


def _sc_gather(src, dst, x_l, x_r, eamask):
    n = x_l.shape[0]
    e = src.shape[0]
    n_iters = e // (_NW * _CH)
    mesh = plsc.VectorSubcoreMesh(core_axis_name="c", subcore_axis_name="s")
    zeros = jnp.zeros((n // 4, 128), jnp.float32)
    return pl.kernel(
        functools.partial(_sc_gather_body, n_iters),
        out_type=[
            jax.ShapeDtypeStruct((e, 128), jnp.float32),
            jax.ShapeDtypeStruct((e, 128), jnp.float32),
            jax.ShapeDtypeStruct((_NC, n // 4, 128), jnp.float32),
        ],
        mesh=mesh,
        scratch_types=[
            [pltpu.VMEM((_CH,), jnp.int32) for _ in range(2)],
            [pltpu.VMEM((_CH,), jnp.int32) for _ in range(2)],
            [pltpu.VMEM((_CH,), jnp.int32) for _ in range(2)],
            [pltpu.VMEM((_CH, 128), jnp.float32) for _ in range(2)],
            [pltpu.VMEM((_CH, 128), jnp.float32) for _ in range(2)],
            [pltpu.VMEM((_CH, 128), jnp.float32) for _ in range(2)],
            [[pltpu.SemaphoreType.DMA for _ in range(2)] for _ in range(2)],
            [[pltpu.SemaphoreType.DMA for _ in range(3)] for _ in range(2)],
            [[pltpu.SemaphoreType.DMA for _ in range(3)] for _ in range(2)],
            pltpu.VMEM_SHARED((n // 4, 128), jnp.float32),
        ],
    )(src, dst, x_l, x_r, eamask, zeros)


def _sc_scatter(dst, w, exrow, n):
    e = dst.shape[0]
    n_iters = e // (_NW * _CH)
    mesh = plsc.VectorSubcoreMesh(core_axis_name="c", subcore_axis_name="s")
    zeros = jnp.zeros((n, 128), jnp.float32)
    zeros_d = jnp.zeros((n // 16, 128), jnp.float32)
    return pl.kernel(
        functools.partial(_sc_scatter_body, n_iters),
        out_type=[
            jax.ShapeDtypeStruct((_NC, n, 128), jnp.float32),
            jax.ShapeDtypeStruct((_NC, n // 16, 128), jnp.float32),
        ],
        mesh=mesh,
        scratch_types=[
            [pltpu.VMEM((_CH,), jnp.int32) for _ in range(2)],
            [pltpu.VMEM((_CH,), jnp.int32) for _ in range(2)],
            [pltpu.VMEM((_CH,), jnp.int32) for _ in range(2)],
            [pltpu.VMEM((_CH, 128), jnp.float32) for _ in range(2)],
            [pltpu.VMEM((_CH, 128), jnp.float32) for _ in range(2)],
            [[pltpu.SemaphoreType.DMA for _ in range(3)] for _ in range(2)],
            [[pltpu.SemaphoreType.DMA for _ in range(2)] for _ in range(2)],
            pltpu.VMEM_SHARED((n, 128), jnp.float32),
            pltpu.VMEM_SHARED((n // 16, 128), jnp.float32),
        ],
    )(dst, w, exrow, zeros, zeros_d)


def kernel(x, edge_index, edge_attr, W_l, b_l, W_r, b_r, W_e, att, bias):
    n, d_in = x.shape
    e = edge_index.shape[1]
    heads, c = att.shape
    hc = heads * c

    src = edge_index[0]
    dst = edge_index[1]

    be = 2000
    bn = 2000
    ne_blocks = e // be
    nn_blocks = n // bn

    src3 = src.reshape(ne_blocks, be, 1)
    dst3 = dst.reshape(ne_blocks, be, 1)

    eye = jnp.eye
    a_att = (att[:, :, None] * eye(heads, dtype=jnp.float32)[:, None, :]
             ).reshape(hc, heads)
    r_rep = jnp.repeat(eye(heads, dtype=jnp.float32), c, axis=1)
    t16 = sum(eye(16, 128, 32 * k, dtype=jnp.float32) for k in range(4))
    t8 = sum(eye(8, 128, 8 * k, dtype=jnp.float32) for k in range(16))

    x_l, x_r = pl.pallas_call(
        _proj_body,
        out_shape=[jax.ShapeDtypeStruct((n, hc), jnp.float32),
                   jax.ShapeDtypeStruct((n, hc), jnp.float32)],
        interpret=_INTERPRET_TC,
    )(x, W_l, b_l, W_r, b_r)

    eamask = pl.pallas_call(
        _eamask_body,
        grid=(ne_blocks,),
        in_specs=[
            pl.BlockSpec((be, 16), lambda i: (i, 0)),
            pl.BlockSpec((1, be, 1), lambda i: (i, 0, 0)),
            pl.BlockSpec((1, be, 1), lambda i: (i, 0, 0)),
            pl.BlockSpec((16, 128), lambda i: (0, 0)),
        ],
        out_specs=pl.BlockSpec((be, 128), lambda i: (i, 0)),
        out_shape=jax.ShapeDtypeStruct((e, 128), jnp.float32),
        interpret=_INTERPRET_TC,
    )(edge_attr, src3, dst3, t16)

    e_emb = pl.pallas_call(
        _eemb_body,
        grid=(ne_blocks,),
        in_specs=[
            pl.BlockSpec((be, 16), lambda i: (i, 0)),
            pl.BlockSpec((16, hc), lambda i: (0, 0)),
        ],
        out_specs=pl.BlockSpec((be, hc), lambda i: (i, 0)),
        out_shape=jax.ShapeDtypeStruct((e, hc), jnp.float32),
        interpret=_INTERPRET_TC,
    )(edge_attr, W_e)

    xl_src, xr_dst, lacc_p = _sc_gather(src, dst, x_l, x_r, eamask)
    lacc = lacc_p.reshape(2, n, 32)

    w, exrow = pl.pallas_call(
        _edge_body,
        grid=(ne_blocks,),
        in_specs=[
            pl.BlockSpec((be, hc), lambda i: (i, 0)),
            pl.BlockSpec((be, hc), lambda i: (i, 0)),
            pl.BlockSpec((be, hc), lambda i: (i, 0)),
            pl.BlockSpec((1, be, 1), lambda i: (i, 0, 0)),
            pl.BlockSpec((1, be, 1), lambda i: (i, 0, 0)),
            pl.BlockSpec((hc, heads), lambda i: (0, 0)),
            pl.BlockSpec((heads, hc), lambda i: (0, 0)),
            pl.BlockSpec((heads, 128), lambda i: (0, 0)),
        ],
        out_specs=[
            pl.BlockSpec((be, hc), lambda i: (i, 0)),
            pl.BlockSpec((be, 128), lambda i: (i, 0)),
        ],
        out_shape=[jax.ShapeDtypeStruct((e, hc), jnp.float32),
                   jax.ShapeDtypeStruct((e, 128), jnp.float32)],
        interpret=_INTERPRET_TC,
    )(xl_src, xr_dst, e_emb, src3, dst3, a_att, r_rep, t8)

    wacc, dacc_p = _sc_scatter(dst, w, exrow, n)
    den = dacc_p.reshape(2, n, heads)

    out = pl.pallas_call(
        _final_body,
        grid=(nn_blocks,),
        in_specs=[
            pl.BlockSpec((2, bn, 32), lambda i: (0, i, 0)),
            pl.BlockSpec((2, bn, hc), lambda i: (0, i, 0)),
            pl.BlockSpec((2, bn, heads), lambda i: (0, i, 0)),
            pl.BlockSpec((bn, hc), lambda i: (i, 0)),
            pl.BlockSpec((bn, hc), lambda i: (i, 0)),
            pl.BlockSpec((16, hc), lambda i: (0, 0)),
            pl.BlockSpec((hc, heads), lambda i: (0, 0)),
            pl.BlockSpec((heads, hc), lambda i: (0, 0)),
            pl.BlockSpec((hc,), lambda i: (0,)),
        ],
        out_specs=pl.BlockSpec((bn, hc), lambda i: (i, 0)),
        out_shape=jax.ShapeDtypeStruct((n, hc), jnp.float32),
        interpret=_INTERPRET_TC,
    )(lacc, wacc, den, x_l, x_r, W_e, a_att, r_rep, bias)

    return out


# speedup vs baseline: 1.0088x; 1.0088x over previous
import functools

import jax
import jax.numpy as jnp
from jax import lax
from jax.experimental import pallas as pl
from jax.experimental.pallas import tpu as pltpu
from jax.experimental.pallas import tpu_sc as plsc

_INTERPRET_TC = False


def _proj_body(x_ref, wl_ref, bl_ref, wr_ref, br_ref, xl_ref, xr_ref):
    x = x_ref[...]
    xl_ref[...] = (jnp.dot(x, wl_ref[...], preferred_element_type=jnp.float32)
                   + bl_ref[...][None, :])
    xr_ref[...] = (jnp.dot(x, wr_ref[...], preferred_element_type=jnp.float32)
                   + br_ref[...][None, :])


def _eamask_body(ea_ref, s_ref, d_ref, t16_ref, eam_ref):
    ea = ea_ref[...]
    s = s_ref[0]
    d = d_ref[0]
    keep = (s != d).astype(jnp.float32)
    b = ea.shape[0]
    col = lax.broadcasted_iota(jnp.int32, (b, 128), 1)
    tiled = jnp.dot(ea, t16_ref[...], preferred_element_type=jnp.float32)
    row = jnp.where(col % 32 == 16, 1.0, tiled)
    kmask = (col // 32 == d % 4).astype(jnp.float32)
    eam_ref[...] = row * keep * kmask


def _eemb_body(ea_ref, we_ref, eemb_ref):
    eemb_ref[...] = jnp.dot(ea_ref[...], we_ref[...],
                            preferred_element_type=jnp.float32)


def _edge_body(xl_ref, xr_ref, ee_ref, s_ref, d_ref, aatt_ref, rrep_ref,
               t8_ref, w_ref, exrow_ref):
    xl = xl_ref[...]
    m = xl + xr_ref[...] + ee_ref[...]
    mlr = jnp.where(m > 0, m, 0.2 * m)
    alpha = jnp.dot(mlr, aatt_ref[...], preferred_element_type=jnp.float32)
    s = s_ref[0]
    d = d_ref[0]
    keep = (s != d).astype(jnp.float32)
    ex = jnp.exp(alpha) * keep
    w_ref[...] = jnp.dot(ex, rrep_ref[...],
                         preferred_element_type=jnp.float32) * xl
    b = xl.shape[0]
    col = lax.broadcasted_iota(jnp.int32, (b, 128), 1)
    tiledex = jnp.dot(ex, t8_ref[...], preferred_element_type=jnp.float32)
    exrow_ref[...] = tiledex * (col // 8 == d % 16).astype(jnp.float32)


def _final_body(lacc_ref, wacc_ref, den_ref, xl_ref, xr_ref, we_ref,
                aatt_ref, rrep_ref, bias_ref, out_ref):
    lsum = lacc_ref[0] + lacc_ref[1]
    deg = lsum[:, 16:17]
    la = lsum[:, 0:16] / jnp.maximum(deg, 1.0)
    el = jnp.dot(la, we_ref[...], preferred_element_type=jnp.float32)
    xl = xl_ref[...]
    m = xl + xr_ref[...] + el
    mlr = jnp.where(m > 0, m, 0.2 * m)
    al = jnp.dot(mlr, aatt_ref[...], preferred_element_type=jnp.float32)
    exl = jnp.exp(al)
    densum = den_ref[0] + den_ref[1] + exl
    rrep = rrep_ref[...]
    num = (wacc_ref[0] + wacc_ref[1]
           + jnp.dot(exl, rrep, preferred_element_type=jnp.float32) * xl)
    denrep = jnp.dot(densum, rrep, preferred_element_type=jnp.float32) + 1e-30
    out_ref[...] = num / denrep + bias_ref[...][None, :]


_NC, _NS = 2, 16
_NW = _NC * _NS
_CH = 80


def _sc_gather_body(n_iters, src_hbm, dst_hbm, xl_hbm, xr_hbm, eam_hbm, z_hbm,
                    xls_hbm, xrd_hbm, lacc_hbm,
                    sidx, didx, didx4, rows_a, rows_b, ea_rows,
                    sem_i, sem_g, sem_w, acc_sp):
    cid = lax.axis_index("c")
    sid = lax.axis_index("s")
    wid = sid * _NC + cid
    epw = n_iters * _CH

    @pl.when(sid == 0)
    def _init():
        pltpu.sync_copy(z_hbm, acc_sp)
    plsc.subcore_barrier()

    def issue_idx(ci, b):
        base = wid * epw + ci * _CH
        return (
            pltpu.async_copy(src_hbm.at[pl.ds(base, _CH)], sidx[b],
                             sem_i[b][0]),
            pltpu.async_copy(dst_hbm.at[pl.ds(base, _CH)], didx[b],
                             sem_i[b][1]),
        )

    def issue_gathers(ci, b):
        base = wid * epw + ci * _CH
        return (
            pltpu.async_copy(xl_hbm.at[sidx[b]], rows_a[b], sem_g[b][0]),
            pltpu.async_copy(xr_hbm.at[didx[b]], rows_b[b], sem_g[b][1]),
            pltpu.async_copy(eam_hbm.at[pl.ds(base, _CH)], ea_rows[b],
                             sem_g[b][2]),
        )

    def issue_drain(ci, b):
        base = wid * epw + ci * _CH
        w1 = pltpu.async_copy(rows_a[b], xls_hbm.at[pl.ds(base, _CH)],
                              sem_w[b][0])
        w2 = pltpu.async_copy(rows_b[b], xrd_hbm.at[pl.ds(base, _CH)],
                              sem_w[b][1])
        for g in range(_CH // 16):
            didx4[b][pl.ds(g * 16, 16)] = didx[b][pl.ds(g * 16, 16)] >> 2
        w3 = pltpu.async_copy(ea_rows[b], acc_sp.at[didx4[b]], sem_w[b][2],
                              add=True)
        return (w1, w2, w3)

    def wait_all(handles):
        for h in handles:
            h.wait()

    def do_group(cs):
        his = [issue_idx(c, b) for b, c in enumerate(cs)]
        hgs = []
        for b, c in enumerate(cs):
            wait_all(his[b])
            hgs.append(issue_gathers(c, b))
        hws = []
        for b, c in enumerate(cs):
            wait_all(hgs[b])
            hws.append(issue_drain(c, b))
        for hw in hws:
            wait_all(hw)

    np_ = len(sidx)

    def groupbody(i, _):
        do_group([np_ * i + b for b in range(np_)])
        return ()

    lax.fori_loop(0, n_iters // np_, groupbody, (), unroll=False)
    for c in range(n_iters - n_iters % np_, n_iters):
        wait_all(issue_idx(c, 0))
        wait_all(issue_gathers(c, 0))
        wait_all(issue_drain(c, 0))

    plsc.subcore_barrier()

    @pl.when(sid == 0)
    def _writeout():
        pltpu.sync_copy(acc_sp, lacc_hbm.at[cid])


def _sc_scatter_body(n_iters, dst_hbm, w_hbm, exrow_hbm, z_hbm, zd_hbm,
                     wacc_hbm, dacc_hbm,
                     didx, didxc, didx16, w_rows, ex_rows, sem_i, sem_s,
                     wacc_sp, den_sp):
    cid = lax.axis_index("c")
    sid = lax.axis_index("s")
    wid = sid * _NC + cid
    epw = n_iters * _CH

    @pl.when(sid == 0)
    def _init_w():
        pltpu.sync_copy(z_hbm, wacc_sp)

    @pl.when(sid == 1)
    def _init_d():
        pltpu.sync_copy(zd_hbm, den_sp)
    plsc.subcore_barrier()

    def issue_loads(ci, b):
        base = wid * epw + ci * _CH
        return (
            pltpu.async_copy(dst_hbm.at[pl.ds(base, _CH)], didx[b],
                             sem_i[b][0]),
            pltpu.async_copy(w_hbm.at[pl.ds(base, _CH)], w_rows[b],
                             sem_i[b][1]),
            pltpu.async_copy(exrow_hbm.at[pl.ds(base, _CH)], ex_rows[b],
                             sem_i[b][2]),
        )

    def issue_scatters(b):
        for g in range(_CH // 16):
            didx16[b][pl.ds(g * 16, 16)] = didx[b][pl.ds(g * 16, 16)] >> 4
        return (
            pltpu.async_copy(w_rows[b], wacc_sp.at[didx[b]], sem_s[b][0],
                             add=True),
            pltpu.async_copy(ex_rows[b], den_sp.at[didx16[b]], sem_s[b][1],
                             add=True),
        )

    def wait_all(handles):
        for h in handles:
            h.wait()

    def do_group(cs):
        hls = [issue_loads(c, b) for b, c in enumerate(cs)]
        hss = []
        for b in range(len(cs)):
            wait_all(hls[b])
            hss.append(issue_scatters(b))
        for hs in hss:
            wait_all(hs)

    np_ = len(didx)

    def groupbody(i, _):
        do_group([np_ * i + b for b in range(np_)])
        return ()

    lax.fori_loop(0, n_iters // np_, groupbody, (), unroll=False)
    for c in range(n_iters - n_iters % np_, n_iters):
        wait_all(issue_loads(c, 0))
        wait_all(issue_scatters(0))
    plsc.subcore_barrier()

    rows_per_tile = wacc_sp.shape[0] // 10

    @pl.when(sid < 10)
    def _writeout_w():
        r0 = sid * rows_per_tile
        pltpu.sync_copy(wacc_sp.at[pl.ds(r0, rows_per_tile)],
                        wacc_hbm.at[cid, pl.ds(r0, rows_per_tile)])

    @pl.when(sid == 10)
    def _writeout_d():
        pltpu.sync_copy(den_sp, dacc_hbm.at[cid])


def _sc_gather(src, dst, x_l, x_r, eamask):
    n = x_l.shape[0]
    e = src.shape[0]
    n_iters = e // (_NW * _CH)
    mesh = plsc.VectorSubcoreMesh(core_axis_name="c", subcore_axis_name="s")
    zeros = jnp.zeros((n // 4, 128), jnp.float32)
    return pl.kernel(
        functools.partial(_sc_gather_body, n_iters),
        out_type=[
            jax.ShapeDtypeStruct((e, 128), jnp.float32),
            jax.ShapeDtypeStruct((e, 128), jnp.float32),
            jax.ShapeDtypeStruct((_NC, n // 4, 128), jnp.float32),
        ],
        mesh=mesh,
        scratch_types=[
            [pltpu.VMEM((_CH,), jnp.int32) for _ in range(3)],
            [pltpu.VMEM((_CH,), jnp.int32) for _ in range(3)],
            [pltpu.VMEM((_CH,), jnp.int32) for _ in range(3)],
            [pltpu.VMEM((_CH, 128), jnp.float32) for _ in range(3)],
            [pltpu.VMEM((_CH, 128), jnp.float32) for _ in range(3)],
            [pltpu.VMEM((_CH, 128), jnp.float32) for _ in range(3)],
            [[pltpu.SemaphoreType.DMA for _ in range(3)] for _ in range(3)],
            [[pltpu.SemaphoreType.DMA for _ in range(3)] for _ in range(3)],
            [[pltpu.SemaphoreType.DMA for _ in range(3)] for _ in range(3)],
            pltpu.VMEM_SHARED((n // 4, 128), jnp.float32),
        ],
    )(src, dst, x_l, x_r, eamask, zeros)


def _sc_scatter(dst, w, exrow, n):
    e = dst.shape[0]
    n_iters = e // (_NW * _CH)
    mesh = plsc.VectorSubcoreMesh(core_axis_name="c", subcore_axis_name="s")
    zeros = jnp.zeros((n, 128), jnp.float32)
    zeros_d = jnp.zeros((n // 16, 128), jnp.float32)
    return pl.kernel(
        functools.partial(_sc_scatter_body, n_iters),
        out_type=[
            jax.ShapeDtypeStruct((_NC, n, 128), jnp.float32),
            jax.ShapeDtypeStruct((_NC, n // 16, 128), jnp.float32),
        ],
        mesh=mesh,
        scratch_types=[
            [pltpu.VMEM((_CH,), jnp.int32) for _ in range(2)],
            [pltpu.VMEM((_CH,), jnp.int32) for _ in range(2)],
            [pltpu.VMEM((_CH,), jnp.int32) for _ in range(2)],
            [pltpu.VMEM((_CH, 128), jnp.float32) for _ in range(2)],
            [pltpu.VMEM((_CH, 128), jnp.float32) for _ in range(2)],
            [[pltpu.SemaphoreType.DMA for _ in range(3)] for _ in range(2)],
            [[pltpu.SemaphoreType.DMA for _ in range(2)] for _ in range(2)],
            pltpu.VMEM_SHARED((n, 128), jnp.float32),
            pltpu.VMEM_SHARED((n // 16, 128), jnp.float32),
        ],
    )(dst, w, exrow, zeros, zeros_d)


def kernel(x, edge_index, edge_attr, W_l, b_l, W_r, b_r, W_e, att, bias):
    n, d_in = x.shape
    e = edge_index.shape[1]
    heads, c = att.shape
    hc = heads * c

    src = edge_index[0]
    dst = edge_index[1]

    be = 2000
    bn = 2000
    ne_blocks = e // be
    nn_blocks = n // bn

    src3 = src.reshape(ne_blocks, be, 1)
    dst3 = dst.reshape(ne_blocks, be, 1)

    eye = jnp.eye
    a_att = (att[:, :, None] * eye(heads, dtype=jnp.float32)[:, None, :]
             ).reshape(hc, heads)
    r_rep = jnp.repeat(eye(heads, dtype=jnp.float32), c, axis=1)
    t16 = sum(eye(16, 128, 32 * k, dtype=jnp.float32) for k in range(4))
    t8 = sum(eye(8, 128, 8 * k, dtype=jnp.float32) for k in range(16))

    x_l, x_r = pl.pallas_call(
        _proj_body,
        out_shape=[jax.ShapeDtypeStruct((n, hc), jnp.float32),
                   jax.ShapeDtypeStruct((n, hc), jnp.float32)],
        interpret=_INTERPRET_TC,
    )(x, W_l, b_l, W_r, b_r)

    eamask = pl.pallas_call(
        _eamask_body,
        grid=(ne_blocks,),
        in_specs=[
            pl.BlockSpec((be, 16), lambda i: (i, 0)),
            pl.BlockSpec((1, be, 1), lambda i: (i, 0, 0)),
            pl.BlockSpec((1, be, 1), lambda i: (i, 0, 0)),
            pl.BlockSpec((16, 128), lambda i: (0, 0)),
        ],
        out_specs=pl.BlockSpec((be, 128), lambda i: (i, 0)),
        out_shape=jax.ShapeDtypeStruct((e, 128), jnp.float32),
        interpret=_INTERPRET_TC,
    )(edge_attr, src3, dst3, t16)

    e_emb = pl.pallas_call(
        _eemb_body,
        grid=(ne_blocks,),
        in_specs=[
            pl.BlockSpec((be, 16), lambda i: (i, 0)),
            pl.BlockSpec((16, hc), lambda i: (0, 0)),
        ],
        out_specs=pl.BlockSpec((be, hc), lambda i: (i, 0)),
        out_shape=jax.ShapeDtypeStruct((e, hc), jnp.float32),
        interpret=_INTERPRET_TC,
    )(edge_attr, W_e)

    xl_src, xr_dst, lacc_p = _sc_gather(src, dst, x_l, x_r, eamask)
    lacc = lacc_p.reshape(2, n, 32)

    w, exrow = pl.pallas_call(
        _edge_body,
        grid=(ne_blocks,),
        in_specs=[
            pl.BlockSpec((be, hc), lambda i: (i, 0)),
            pl.BlockSpec((be, hc), lambda i: (i, 0)),
            pl.BlockSpec((be, hc), lambda i: (i, 0)),
            pl.BlockSpec((1, be, 1), lambda i: (i, 0, 0)),
            pl.BlockSpec((1, be, 1), lambda i: (i, 0, 0)),
            pl.BlockSpec((hc, heads), lambda i: (0, 0)),
            pl.BlockSpec((heads, hc), lambda i: (0, 0)),
            pl.BlockSpec((heads, 128), lambda i: (0, 0)),
        ],
        out_specs=[
            pl.BlockSpec((be, hc), lambda i: (i, 0)),
            pl.BlockSpec((be, 128), lambda i: (i, 0)),
        ],
        out_shape=[jax.ShapeDtypeStruct((e, hc), jnp.float32),
                   jax.ShapeDtypeStruct((e, 128), jnp.float32)],
        interpret=_INTERPRET_TC,
    )(xl_src, xr_dst, e_emb, src3, dst3, a_att, r_rep, t8)

    wacc, dacc_p = _sc_scatter(dst, w, exrow, n)
    den = dacc_p.reshape(2, n, heads)

    out = pl.pallas_call(
        _final_body,
        grid=(nn_blocks,),
        in_specs=[
            pl.BlockSpec((2, bn, 32), lambda i: (0, i, 0)),
            pl.BlockSpec((2, bn, hc), lambda i: (0, i, 0)),
            pl.BlockSpec((2, bn, heads), lambda i: (0, i, 0)),
            pl.BlockSpec((bn, hc), lambda i: (i, 0)),
            pl.BlockSpec((bn, hc), lambda i: (i, 0)),
            pl.BlockSpec((16, hc), lambda i: (0, 0)),
            pl.BlockSpec((hc, heads), lambda i: (0, 0)),
            pl.BlockSpec((heads, hc), lambda i: (0, 0)),
            pl.BlockSpec((hc,), lambda i: (0,)),
        ],
        out_specs=pl.BlockSpec((bn, hc), lambda i: (i, 0)),
        out_shape=jax.ShapeDtypeStruct((n, hc), jnp.float32),
        interpret=_INTERPRET_TC,
    )(lacc, wacc, den, x_l, x_r, W_e, a_att, r_rep, bias)

    return out


# speedup vs baseline: 1.0307x; 1.0217x over previous
import functools

import jax
import jax.numpy as jnp
from jax import lax
from jax.experimental import pallas as pl
from jax.experimental.pallas import tpu as pltpu
from jax.experimental.pallas import tpu_sc as plsc


def _proj_body(x_ref, wl_ref, bl_ref, wr_ref, br_ref, xl_ref, xr_ref):
    x = x_ref[...]
    xl_ref[...] = (jnp.dot(x, wl_ref[...], preferred_element_type=jnp.float32)
                   + bl_ref[...][None, :])
    xr_ref[...] = (jnp.dot(x, wr_ref[...], preferred_element_type=jnp.float32)
                   + br_ref[...][None, :])


def _edgeprep_body(ea_ref, we_ref, s_ref, d_ref, t16_ref,
                   eemb_ref, eam_ref):
    ea = ea_ref[...]
    s = s_ref[0]
    d = d_ref[0]
    keep = (s != d).astype(jnp.float32)
    eemb_ref[...] = jnp.dot(ea, we_ref[...],
                            preferred_element_type=jnp.float32)
    b = ea.shape[0]
    col = lax.broadcasted_iota(jnp.int32, (b, 128), 1)
    tiled = jnp.dot(ea, t16_ref[...], preferred_element_type=jnp.float32)
    row = jnp.where(col % 32 == 16, 1.0, tiled)
    kmask = (col // 32 == d % 4).astype(jnp.float32)
    eam_ref[...] = row * keep * kmask


def _edge_body(xl_ref, xr_ref, ee_ref, s_ref, d_ref, aatt_ref, rrep_ref,
               t8_ref, w_ref, exrow_ref):
    xl = xl_ref[...]
    m = xl + xr_ref[...] + ee_ref[...]
    mlr = jnp.where(m > 0, m, 0.2 * m)
    alpha = jnp.dot(mlr, aatt_ref[...], preferred_element_type=jnp.float32)
    s = s_ref[0]
    d = d_ref[0]
    keep = (s != d).astype(jnp.float32)
    ex = jnp.exp(alpha) * keep
    w_ref[...] = jnp.dot(ex, rrep_ref[...],
                         preferred_element_type=jnp.float32) * xl
    b = xl.shape[0]
    col = lax.broadcasted_iota(jnp.int32, (b, 128), 1)
    tiledex = jnp.dot(ex, t8_ref[...], preferred_element_type=jnp.float32)
    exrow_ref[...] = tiledex * (col // 8 == d % 16).astype(jnp.float32)


def _final_body(lacc_ref, wacc_ref, den_ref, xl_ref, xr_ref, we_ref,
                aatt_ref, rrep_ref, bias_ref, out_ref):
    lsum = lacc_ref[0] + lacc_ref[1]
    deg = lsum[:, 16:17]
    la = lsum[:, 0:16] / jnp.maximum(deg, 1.0)
    el = jnp.dot(la, we_ref[...], preferred_element_type=jnp.float32)
    xl = xl_ref[...]
    m = xl + xr_ref[...] + el
    mlr = jnp.where(m > 0, m, 0.2 * m)
    al = jnp.dot(mlr, aatt_ref[...], preferred_element_type=jnp.float32)
    exl = jnp.exp(al)
    densum = den_ref[0] + den_ref[1] + exl
    rrep = rrep_ref[...]
    num = (wacc_ref[0] + wacc_ref[1]
           + jnp.dot(exl, rrep, preferred_element_type=jnp.float32) * xl)
    denrep = jnp.dot(densum, rrep, preferred_element_type=jnp.float32) + 1e-30
    out_ref[...] = num / denrep + bias_ref[...][None, :]


_NC, _NS = 2, 16
_NW = _NC * _NS
_CH = 80


def _sc_gather_body(n_iters, src_hbm, dst_hbm, xl_hbm, xr_hbm, eam_hbm, z_hbm,
                    xls_hbm, xrd_hbm, lacc_hbm,
                    sidx, didx, didx4, rows_a, rows_b, ea_rows,
                    sem_i, sem_g, sem_w, acc_sp):
    cid = lax.axis_index("c")
    sid = lax.axis_index("s")
    wid = sid * _NC + cid
    epw = n_iters * _CH

    @pl.when(sid == 0)
    def _init():
        pltpu.sync_copy(z_hbm, acc_sp)
    plsc.subcore_barrier()

    def issue_idx(ci, b):
        base = wid * epw + ci * _CH
        return (
            pltpu.async_copy(src_hbm.at[pl.ds(base, _CH)], sidx[b],
                             sem_i[b][0]),
            pltpu.async_copy(dst_hbm.at[pl.ds(base, _CH)], didx[b],
                             sem_i[b][1]),
        )

    def issue_gathers(ci, b):
        base = wid * epw + ci * _CH
        return (
            pltpu.async_copy(xl_hbm.at[sidx[b]], rows_a[b], sem_g[b][0]),
            pltpu.async_copy(xr_hbm.at[didx[b]], rows_b[b], sem_g[b][1]),
            pltpu.async_copy(eam_hbm.at[pl.ds(base, _CH)], ea_rows[b],
                             sem_g[b][2]),
        )

    def issue_drain(ci, b):
        base = wid * epw + ci * _CH
        w1 = pltpu.async_copy(rows_a[b], xls_hbm.at[pl.ds(base, _CH)],
                              sem_w[b][0])
        w2 = pltpu.async_copy(rows_b[b], xrd_hbm.at[pl.ds(base, _CH)],
                              sem_w[b][1])
        for g in range(_CH // 16):
            didx4[b][pl.ds(g * 16, 16)] = didx[b][pl.ds(g * 16, 16)] >> 2
        w3 = pltpu.async_copy(ea_rows[b], acc_sp.at[didx4[b]], sem_w[b][2],
                              add=True)
        return (w1, w2, w3)

    def wait_all(handles):
        for h in handles:
            h.wait()

    def do_group(cs):
        his = [issue_idx(c, b) for b, c in enumerate(cs)]
        hgs = []
        for b, c in enumerate(cs):
            wait_all(his[b])
            hgs.append(issue_gathers(c, b))
        hws = []
        for b, c in enumerate(cs):
            wait_all(hgs[b])
            hws.append(issue_drain(c, b))
        for hw in hws:
            wait_all(hw)

    np_ = len(sidx)

    def groupbody(i, _):
        do_group([np_ * i + b for b in range(np_)])
        return ()

    lax.fori_loop(0, n_iters // np_, groupbody, (), unroll=False)
    for c in range(n_iters - n_iters % np_, n_iters):
        wait_all(issue_idx(c, 0))
        wait_all(issue_gathers(c, 0))
        wait_all(issue_drain(c, 0))

    plsc.subcore_barrier()

    @pl.when(sid == 0)
    def _writeout():
        pltpu.sync_copy(acc_sp, lacc_hbm.at[cid])


def _sc_scatter_body(n_iters, dst_hbm, w_hbm, exrow_hbm, z_hbm, zd_hbm,
                     wacc_hbm, dacc_hbm,
                     didx, didxc, didx16, w_rows, ex_rows, sem_i, sem_s,
                     wacc_sp, den_sp):
    cid = lax.axis_index("c")
    sid = lax.axis_index("s")
    wid = sid * _NC + cid
    epw = n_iters * _CH

    @pl.when(sid == 0)
    def _init_w():
        pltpu.sync_copy(z_hbm, wacc_sp)

    @pl.when(sid == 1)
    def _init_d():
        pltpu.sync_copy(zd_hbm, den_sp)
    plsc.subcore_barrier()

    def issue_loads(ci, b):
        base = wid * epw + ci * _CH
        return (
            pltpu.async_copy(dst_hbm.at[pl.ds(base, _CH)], didx[b],
                             sem_i[b][0]),
            pltpu.async_copy(w_hbm.at[pl.ds(base, _CH)], w_rows[b],
                             sem_i[b][1]),
            pltpu.async_copy(exrow_hbm.at[pl.ds(base, _CH)], ex_rows[b],
                             sem_i[b][2]),
        )

    def issue_scatters(b):
        for g in range(_CH // 16):
            didx16[b][pl.ds(g * 16, 16)] = didx[b][pl.ds(g * 16, 16)] >> 4
        return (
            pltpu.async_copy(w_rows[b], wacc_sp.at[didx[b]], sem_s[b][0],
                             add=True),
            pltpu.async_copy(ex_rows[b], den_sp.at[didx16[b]], sem_s[b][1],
                             add=True),
        )

    def wait_all(handles):
        for h in handles:
            h.wait()

    def do_group(cs):
        hls = [issue_loads(c, b) for b, c in enumerate(cs)]
        hss = []
        for b in range(len(cs)):
            wait_all(hls[b])
            hss.append(issue_scatters(b))
        for hs in hss:
            wait_all(hs)

    np_ = len(didx)

    def groupbody(i, _):
        do_group([np_ * i + b for b in range(np_)])
        return ()

    lax.fori_loop(0, n_iters // np_, groupbody, (), unroll=False)
    for c in range(n_iters - n_iters % np_, n_iters):
        wait_all(issue_loads(c, 0))
        wait_all(issue_scatters(0))
    plsc.subcore_barrier()

    rows_per_tile = wacc_sp.shape[0] // 10

    @pl.when(sid < 10)
    def _writeout_w():
        r0 = sid * rows_per_tile
        pltpu.sync_copy(wacc_sp.at[pl.ds(r0, rows_per_tile)],
                        wacc_hbm.at[cid, pl.ds(r0, rows_per_tile)])

    @pl.when(sid == 10)
    def _writeout_d():
        pltpu.sync_copy(den_sp, dacc_hbm.at[cid])


def _sc_gather(src, dst, x_l, x_r, eamask):
    n = x_l.shape[0]
    e = src.shape[0]
    n_iters = e // (_NW * _CH)
    mesh = plsc.VectorSubcoreMesh(core_axis_name="c", subcore_axis_name="s")
    zeros = jnp.zeros((n // 4, 128), jnp.float32)
    return pl.kernel(
        functools.partial(_sc_gather_body, n_iters),
        out_type=[
            jax.ShapeDtypeStruct((e, 128), jnp.float32),
            jax.ShapeDtypeStruct((e, 128), jnp.float32),
            jax.ShapeDtypeStruct((_NC, n // 4, 128), jnp.float32),
        ],
        mesh=mesh,
        scratch_types=[
            [pltpu.VMEM((_CH,), jnp.int32) for _ in range(3)],
            [pltpu.VMEM((_CH,), jnp.int32) for _ in range(3)],
            [pltpu.VMEM((_CH,), jnp.int32) for _ in range(3)],
            [pltpu.VMEM((_CH, 128), jnp.float32) for _ in range(3)],
            [pltpu.VMEM((_CH, 128), jnp.float32) for _ in range(3)],
            [pltpu.VMEM((_CH, 128), jnp.float32) for _ in range(3)],
            [[pltpu.SemaphoreType.DMA for _ in range(3)] for _ in range(3)],
            [[pltpu.SemaphoreType.DMA for _ in range(3)] for _ in range(3)],
            [[pltpu.SemaphoreType.DMA for _ in range(3)] for _ in range(3)],
            pltpu.VMEM_SHARED((n // 4, 128), jnp.float32),
        ],
    )(src, dst, x_l, x_r, eamask, zeros)


def _sc_scatter(dst, w, exrow, n):
    e = dst.shape[0]
    n_iters = e // (_NW * _CH)
    mesh = plsc.VectorSubcoreMesh(core_axis_name="c", subcore_axis_name="s")
    zeros = jnp.zeros((n, 128), jnp.float32)
    zeros_d = jnp.zeros((n // 16, 128), jnp.float32)
    return pl.kernel(
        functools.partial(_sc_scatter_body, n_iters),
        out_type=[
            jax.ShapeDtypeStruct((_NC, n, 128), jnp.float32),
            jax.ShapeDtypeStruct((_NC, n // 16, 128), jnp.float32),
        ],
        mesh=mesh,
        scratch_types=[
            [pltpu.VMEM((_CH,), jnp.int32) for _ in range(2)],
            [pltpu.VMEM((_CH,), jnp.int32) for _ in range(2)],
            [pltpu.VMEM((_CH,), jnp.int32) for _ in range(2)],
            [pltpu.VMEM((_CH, 128), jnp.float32) for _ in range(2)],
            [pltpu.VMEM((_CH, 128), jnp.float32) for _ in range(2)],
            [[pltpu.SemaphoreType.DMA for _ in range(3)] for _ in range(2)],
            [[pltpu.SemaphoreType.DMA for _ in range(2)] for _ in range(2)],
            pltpu.VMEM_SHARED((n, 128), jnp.float32),
            pltpu.VMEM_SHARED((n // 16, 128), jnp.float32),
        ],
    )(dst, w, exrow, zeros, zeros_d)


def kernel(x, edge_index, edge_attr, W_l, b_l, W_r, b_r, W_e, att, bias):
    n, d_in = x.shape
    e = edge_index.shape[1]
    heads, c = att.shape
    hc = heads * c

    src = edge_index[0]
    dst = edge_index[1]

    be = 2000
    bn = 2000
    ne_blocks = e // be
    nn_blocks = n // bn

    src3 = src.reshape(ne_blocks, be, 1)
    dst3 = dst.reshape(ne_blocks, be, 1)

    eye = jnp.eye
    a_att = (att[:, :, None] * eye(heads, dtype=jnp.float32)[:, None, :]
             ).reshape(hc, heads)
    r_rep = jnp.repeat(eye(heads, dtype=jnp.float32), c, axis=1)
    t16 = sum(eye(16, 128, 32 * k, dtype=jnp.float32) for k in range(4))
    t8 = sum(eye(8, 128, 8 * k, dtype=jnp.float32) for k in range(16))

    x_l, x_r = pl.pallas_call(
        _proj_body,
        out_shape=[jax.ShapeDtypeStruct((n, hc), jnp.float32),
                   jax.ShapeDtypeStruct((n, hc), jnp.float32)],
    )(x, W_l, b_l, W_r, b_r)

    e_emb, eamask = pl.pallas_call(
        _edgeprep_body,
        grid=(ne_blocks,),
        in_specs=[
            pl.BlockSpec((be, 16), lambda i: (i, 0)),
            pl.BlockSpec((16, hc), lambda i: (0, 0)),
            pl.BlockSpec((1, be, 1), lambda i: (i, 0, 0)),
            pl.BlockSpec((1, be, 1), lambda i: (i, 0, 0)),
            pl.BlockSpec((16, 128), lambda i: (0, 0)),
        ],
        out_specs=[
            pl.BlockSpec((be, hc), lambda i: (i, 0)),
            pl.BlockSpec((be, 128), lambda i: (i, 0)),
        ],
        out_shape=[jax.ShapeDtypeStruct((e, hc), jnp.float32),
                   jax.ShapeDtypeStruct((e, 128), jnp.float32)],
    )(edge_attr, W_e, src3, dst3, t16)

    xl_src, xr_dst, lacc_p = _sc_gather(src, dst, x_l, x_r, eamask)
    lacc = lacc_p.reshape(2, n, 32)

    w, exrow = pl.pallas_call(
        _edge_body,
        grid=(ne_blocks,),
        in_specs=[
            pl.BlockSpec((be, hc), lambda i: (i, 0)),
            pl.BlockSpec((be, hc), lambda i: (i, 0)),
            pl.BlockSpec((be, hc), lambda i: (i, 0)),
            pl.BlockSpec((1, be, 1), lambda i: (i, 0, 0)),
            pl.BlockSpec((1, be, 1), lambda i: (i, 0, 0)),
            pl.BlockSpec((hc, heads), lambda i: (0, 0)),
            pl.BlockSpec((heads, hc), lambda i: (0, 0)),
            pl.BlockSpec((heads, 128), lambda i: (0, 0)),
        ],
        out_specs=[
            pl.BlockSpec((be, hc), lambda i: (i, 0)),
            pl.BlockSpec((be, 128), lambda i: (i, 0)),
        ],
        out_shape=[jax.ShapeDtypeStruct((e, hc), jnp.float32),
                   jax.ShapeDtypeStruct((e, 128), jnp.float32)],
    )(xl_src, xr_dst, e_emb, src3, dst3, a_att, r_rep, t8)

    wacc, dacc_p = _sc_scatter(dst, w, exrow, n)
    den = dacc_p.reshape(2, n, heads)

    out = pl.pallas_call(
        _final_body,
        grid=(nn_blocks,),
        in_specs=[
            pl.BlockSpec((2, bn, 32), lambda i: (0, i, 0)),
            pl.BlockSpec((2, bn, hc), lambda i: (0, i, 0)),
            pl.BlockSpec((2, bn, heads), lambda i: (0, i, 0)),
            pl.BlockSpec((bn, hc), lambda i: (i, 0)),
            pl.BlockSpec((bn, hc), lambda i: (i, 0)),
            pl.BlockSpec((16, hc), lambda i: (0, 0)),
            pl.BlockSpec((hc, heads), lambda i: (0, 0)),
            pl.BlockSpec((heads, hc), lambda i: (0, 0)),
            pl.BlockSpec((hc,), lambda i: (0,)),
        ],
        out_specs=pl.BlockSpec((bn, hc), lambda i: (i, 0)),
        out_shape=jax.ShapeDtypeStruct((n, hc), jnp.float32),
    )(lacc, wacc, den, x_l, x_r, W_e, a_att, r_rep, bias)

    return out
